# baseline (speedup 1.0000x reference)
import functools

import jax
import jax.numpy as jnp
from jax import lax
from jax.experimental import pallas as pl
from jax.experimental.pallas import tpu as pltpu
from jax.experimental.pallas import tpu_sc as plsc

N = 10000
E = 320000
D = 128
B = 100000
BPAD = 102400
NC, NS = 2, 16
EPT = E // NS
G = 100
NG = EPT // G
RPT = 624
RPT_LAST = N - 15 * RPT
CW = 16
LG = 80
LPW = BPAD // (NC * NS)
LNG = LPW // LG
RB = 1000
SB = 1024

_mesh = plsc.VectorSubcoreMesh(core_axis_name="c", subcore_axis_name="s")


NCH = 10
CHG = NG // NCH

_sc_params = pltpu.CompilerParams(use_tc_tiling_on_sc=False)


def _make_spmm(with_counts):
  out_type = [jax.ShapeDtypeStruct((2, N, D), jnp.float32)]
  scratch = [
      pltpu.VMEM((CHG, G), jnp.int32),
      pltpu.VMEM((CHG, G), jnp.int32),
      pltpu.VMEM((CHG, G), jnp.int32),
      pltpu.VMEM((CHG, G), jnp.int32),
      pltpu.VMEM((G, D), jnp.float32),
      pltpu.VMEM((G, D), jnp.float32),
      pltpu.VMEM_SHARED((N, D), jnp.float32),
      pltpu.SemaphoreType.DMA,
      pltpu.SemaphoreType.DMA,
      pltpu.SemaphoreType.DMA,
      pltpu.SemaphoreType.DMA,
  ]
  if with_counts:
    out_type.append(jax.ShapeDtypeStruct((2, N, CW), jnp.float32))
    scratch += [
        pltpu.VMEM((G, CW), jnp.float32),
        pltpu.VMEM_SHARED((N, CW), jnp.float32),
    ]

  @functools.partial(pl.kernel, out_type=out_type, mesh=_mesh,
                     scratch_types=scratch, compiler_params=_sc_params)
  def spmm(*refs):
    if with_counts:
      (t_cat, e_src, e_dst, zrow, zcnt, ones_h,
       agg_out, cnt_out, src_v0, src_v1, dst_v0, dst_v1, rows0, rows1, acc,
       sem0, sem1, semi0, semi1, ones_v, cacc) = refs
    else:
      (t_cat, e_src, e_dst, zrow,
       agg_out, src_v0, src_v1, dst_v0, dst_v1, rows0, rows1, acc,
       sem0, sem1, semi0, semi1) = refs
    src_bufs, dst_bufs = (src_v0, src_v1), (dst_v0, dst_v1)
    c = lax.axis_index("c")
    s = lax.axis_index("s")
    row0 = s * RPT

    def per_tile(fn):
      @pl.when(s < NS - 1)
      def _():
        fn(row0, RPT)

      @pl.when(s == NS - 1)
      def _():
        fn((NS - 1) * RPT, RPT_LAST)

    per_tile(lambda o, n: pltpu.sync_copy(zrow.at[pl.ds(0, n)],
                                          acc.at[pl.ds(o, n)]))
    if with_counts:
      per_tile(lambda o, n: pltpu.sync_copy(zcnt.at[pl.ds(0, n)],
                                            cacc.at[pl.ds(o, n)]))
      pltpu.sync_copy(ones_h, ones_v)
    plsc.subcore_barrier()

    def accum(dst_v, rows, g):
      pltpu.sync_copy(rows, acc.at[dst_v.at[g]], add=True)
      if with_counts:
        pltpu.sync_copy(ones_v, cacc.at[dst_v.at[g]], add=True)

    def make_body(p):
      src_v, dst_v = src_bufs[p], dst_bufs[p]

      def body(j, carry):
        g0 = 2 * j
        h1 = pltpu.async_copy(t_cat.at[src_v.at[g0 + 1]], rows1, sem1)
        pltpu.make_async_copy(t_cat.at[src_v.at[g0]], rows0, sem0).wait()
        accum(dst_v, rows0, g0)

        @pl.when(j < CHG // 2 - 1)
        def _():
          pltpu.async_copy(t_cat.at[src_v.at[g0 + 2]], rows0, sem0)
        h1.wait()
        accum(dst_v, rows1, g0 + 1)
        return carry
      return body

    pltpu.sync_copy(e_src.at[c, s, pl.ds(0, CHG)], src_bufs[0])
    pltpu.sync_copy(e_dst.at[c, s, pl.ds(0, CHG)], dst_bufs[0])
    for ci in range(NCH):
      p = ci % 2
      if ci + 1 < NCH:
        hs = pltpu.async_copy(e_src.at[c, s, pl.ds((ci + 1) * CHG, CHG)],
                              src_bufs[1 - p], semi0)
        hd = pltpu.async_copy(e_dst.at[c, s, pl.ds((ci + 1) * CHG, CHG)],
                              dst_bufs[1 - p], semi1)
      pltpu.async_copy(t_cat.at[src_bufs[p].at[0]], rows0, sem0)
      lax.fori_loop(0, CHG // 2, make_body(p), 0)
      if ci + 1 < NCH:
        hs.wait()
        hd.wait()

    plsc.subcore_barrier()
    per_tile(lambda o, n: pltpu.sync_copy(acc.at[pl.ds(o, n)],
                                          agg_out.at[c, pl.ds(o, n)]))
    if with_counts:
      per_tile(lambda o, n: pltpu.sync_copy(cacc.at[pl.ds(o, n)],
                                            cnt_out.at[c, pl.ds(o, n)]))

  return spmm


_spmm_counts = _make_spmm(True)
_spmm_plain = _make_spmm(False)


@functools.partial(
    pl.kernel,
    out_type=jax.ShapeDtypeStruct((2, BPAD, D), jnp.float32),
    mesh=_mesh,
    scratch_types=[
        pltpu.VMEM((LNG, LG), jnp.int32),
        pltpu.VMEM((LG, D), jnp.float32),
        pltpu.VMEM((LG, D), jnp.float32),
        pltpu.SemaphoreType.DMA,
        pltpu.SemaphoreType.DMA,
    ])
def _link_gather(u_cat, idx, z_out, idx_v, rows0, rows1, sem0, sem1):
  c = lax.axis_index("c")
  s = lax.axis_index("s")
  w = s * NC + c
  for t in range(2):
    pltpu.sync_copy(idx.at[t, w], idx_v)
    pltpu.async_copy(u_cat.at[idx_v.at[0]], rows0, sem0)

    def body(j, carry):
      g0 = 2 * j
      h1 = pltpu.async_copy(u_cat.at[idx_v.at[g0 + 1]], rows1, sem1)
      pltpu.make_async_copy(u_cat.at[idx_v.at[g0]], rows0, sem0).wait()
      pltpu.sync_copy(rows0, z_out.at[t, pl.ds(w * LPW + g0 * LG, LG)])

      @pl.when(j < LNG // 2 - 1)
      def _():
        pltpu.async_copy(u_cat.at[idx_v.at[g0 + 2]], rows0, sem0)
      h1.wait()
      pltpu.sync_copy(rows1, z_out.at[t, pl.ds(w * LPW + (g0 + 1) * LG, LG)])
      return carry
    lax.fori_loop(0, LNG // 2, body, 0)


def _pre0_body(xd, xs, w0, w1, t_ref):
  t_ref[0] = jnp.dot(xd[...], w0[...], preferred_element_type=jnp.float32)
  t_ref[1] = jnp.dot(xs[...], w1[...], preferred_element_type=jnp.float32)


def _finish(agg, cnt, rt):
  mean = agg / jnp.maximum(cnt[:, 0:1], 1.0)
  o = mean + rt
  nrm = jnp.sqrt(jnp.sum(o * o, axis=-1, keepdims=True))
  return o / jnp.maximum(nrm, 1e-12)


def _post0_body(agg, cnt, xd, xs, wr0a, wr0b, b0a, b0b,
                wn1a, wn1b, wr1a, wr1b, b1a, b1b, t1_ref, r1_ref):
  f32 = jnp.float32
  di1 = _finish(agg[0], cnt[0],
                jnp.dot(xs[...], wr0a[...], preferred_element_type=f32)
                + b0a[...])
  dr1 = _finish(agg[1], cnt[1],
                jnp.dot(xd[...], wr0b[...], preferred_element_type=f32)
                + b0b[...])
  di1 = jnp.maximum(di1, 0.0)
  dr1 = jnp.maximum(dr1, 0.0)
  t1_ref[0] = jnp.dot(dr1, wn1a[...], preferred_element_type=f32)
  t1_ref[1] = jnp.dot(di1, wn1b[...], preferred_element_type=f32)
  r1_ref[0] = jnp.dot(di1, wr1a[...], preferred_element_type=f32) + b1a[...]
  r1_ref[1] = jnp.dot(dr1, wr1b[...], preferred_element_type=f32) + b1b[...]


def _post1_body(agg, cnt, r1, wt, wb, bl, u_ref):
  f32 = jnp.float32
  di2 = _finish(agg[0], cnt[0], r1[0])
  dr2 = _finish(agg[1], cnt[1], r1[1])
  u_ref[0] = jnp.dot(dr2, wt[...], preferred_element_type=f32) + bl[...]
  u_ref[1] = jnp.dot(di2, wb[...], preferred_element_type=f32)


def _score_body(z, w2, b2, o_ref):
  sc = jnp.sum(jnp.maximum(z[0] + z[1], 0.0) * w2[...], axis=-1)
  o_ref[...] = sc.reshape(SB // D, D) + b2[...]


def _node_spec(stacked):
  if stacked:
    return pl.BlockSpec((2, RB, D), lambda i: (0, i, 0))
  return pl.BlockSpec((RB, D), lambda i: (i, 0))


def _full_spec(shape):
  return pl.BlockSpec(shape, lambda i: tuple(0 for _ in shape))


def kernel(x_drug, x_disease, edge_d2s, edge_s2d, edge_label_index,
           W_nb_0_d2s, b_nb_0_d2s, W_rt_0_d2s,
           W_nb_0_s2d, b_nb_0_s2d, W_rt_0_s2d,
           W_nb_1_d2s, b_nb_1_d2s, W_rt_1_d2s,
           W_nb_1_s2d, b_nb_1_s2d, W_rt_1_s2d,
           W_lin1, b_lin1, W_lin2, b_lin2):
  f32 = jnp.float32
  grid_n = N // RB

  t0 = pl.pallas_call(
      _pre0_body,
      grid=(grid_n,),
      in_specs=[_node_spec(False), _node_spec(False),
                _full_spec((D, D)), _full_spec((D, D))],
      out_specs=_node_spec(True),
      out_shape=jax.ShapeDtypeStruct((2, N, D), f32),
  )(x_drug, x_disease, W_nb_0_d2s, W_nb_0_s2d)

  src_cat = jnp.stack([edge_d2s[0], edge_s2d[0] + N]).reshape(2, NS, NG, G)
  dst_all = jnp.stack([edge_d2s[1], edge_s2d[1]]).reshape(2, NS, NG, G)
  zrow = jnp.zeros((RPT_LAST, D), f32)
  zcnt = jnp.zeros((RPT_LAST, CW), f32)
  ones_g = jnp.ones((G, CW), f32)

  agg0, cnt = _spmm_counts(t0.reshape(2 * N, D), src_cat, dst_all,
                           zrow, zcnt, ones_g)

  b_shapes = lambda b: b.reshape(1, D)
  t1, r1 = pl.pallas_call(
      _post0_body,
      grid=(grid_n,),
      in_specs=[_node_spec(True),
                pl.BlockSpec((2, RB, CW), lambda i: (0, i, 0)),
                _node_spec(False), _node_spec(False),
                _full_spec((D, D)), _full_spec((D, D)),
                _full_spec((1, D)), _full_spec((1, D)),
                _full_spec((D, D)), _full_spec((D, D)),
                _full_spec((D, D)), _full_spec((D, D)),
                _full_spec((1, D)), _full_spec((1, D))],
      out_specs=[_node_spec(True), _node_spec(True)],
      out_shape=[jax.ShapeDtypeStruct((2, N, D), f32),
                 jax.ShapeDtypeStruct((2, N, D), f32)],
  )(agg0, cnt, x_drug, x_disease,
    W_rt_0_d2s, W_rt_0_s2d, b_shapes(b_nb_0_d2s), b_shapes(b_nb_0_s2d),
    W_nb_1_d2s, W_nb_1_s2d, W_rt_1_d2s, W_rt_1_s2d,
    b_shapes(b_nb_1_d2s), b_shapes(b_nb_1_s2d))

  agg1, = _spmm_plain(t1.reshape(2 * N, D), src_cat, dst_all, zrow)

  u = pl.pallas_call(
      _post1_body,
      grid=(grid_n,),
      in_specs=[_node_spec(True),
                pl.BlockSpec((2, RB, CW), lambda i: (0, i, 0)),
                _node_spec(True),
                _full_spec((D, D)), _full_spec((D, D)), _full_spec((1, D))],
      out_specs=_node_spec(True),
      out_shape=jax.ShapeDtypeStruct((2, N, D), f32),
  )(agg1, cnt, r1, W_lin1[:D], W_lin1[D:], b_lin1.reshape(1, D))

  npad = BPAD - B
  row_pad = jnp.concatenate([edge_label_index[0],
                             jnp.arange(npad, dtype=jnp.int32)])
  col_pad = jnp.concatenate([edge_label_index[1],
                             jnp.arange(npad, dtype=jnp.int32)])
  idx_link = jnp.stack([row_pad, col_pad + N]).reshape(2, NC * NS, LNG, LG)
  z = _link_gather(u.reshape(2 * N, D), idx_link)

  scores = pl.pallas_call(
      _score_body,
      grid=(BPAD // SB,),
      in_specs=[pl.BlockSpec((2, SB, D), lambda i: (0, i, 0)),
                _full_spec((1, D)), _full_spec((1, D))],
      out_specs=pl.BlockSpec((SB // D, D), lambda i: (i, 0)),
      out_shape=jax.ShapeDtypeStruct((BPAD // D, D), f32),
  )(z, W_lin2.reshape(1, D), jnp.broadcast_to(b_lin2.reshape(1, 1), (1, D)))

  return scores.reshape(-1)[:B]


# speedup vs baseline: 1.0071x; 1.0071x over previous
import functools

import jax
import jax.numpy as jnp
from jax import lax
from jax.experimental import pallas as pl
from jax.experimental.pallas import tpu as pltpu
from jax.experimental.pallas import tpu_sc as plsc

N = 10000
E = 320000
D = 128
B = 100000
BPAD = 102400
NC, NS = 2, 16
EPT = E // NS
G = 100
NG = EPT // G
RPT = 624
RPT_LAST = N - 15 * RPT
CW = 16
LG = 80
LPW = BPAD // (NC * NS)
LNG = LPW // LG
RB = 1000
SB = 1024

_mesh = plsc.VectorSubcoreMesh(core_axis_name="c", subcore_axis_name="s")


NCH = 5
CHG = NG // NCH

_sc_params = pltpu.CompilerParams(use_tc_tiling_on_sc=False)


def _make_spmm(with_counts):
  out_type = [jax.ShapeDtypeStruct((2, N, D), jnp.float32)]
  scratch = [
      pltpu.VMEM((CHG, G), jnp.int32),
      pltpu.VMEM((CHG, G), jnp.int32),
      pltpu.VMEM((G, D), jnp.float32),
      pltpu.VMEM((G, D), jnp.float32),
      pltpu.VMEM_SHARED((N, D), jnp.float32),
      pltpu.SemaphoreType.DMA,
      pltpu.SemaphoreType.DMA,
  ]
  if with_counts:
    out_type.append(jax.ShapeDtypeStruct((2, N, CW), jnp.float32))
    scratch += [
        pltpu.VMEM((G, CW), jnp.float32),
        pltpu.VMEM_SHARED((N, CW), jnp.float32),
    ]

  @functools.partial(pl.kernel, out_type=out_type, mesh=_mesh,
                     scratch_types=scratch, compiler_params=_sc_params)
  def spmm(*refs):
    if with_counts:
      (t_cat, e_src, e_dst, zrow, zcnt, ones_h,
       agg_out, cnt_out, src_v, dst_v, rows0, rows1, acc,
       sem0, sem1, ones_v, cacc) = refs
    else:
      (t_cat, e_src, e_dst, zrow,
       agg_out, src_v, dst_v, rows0, rows1, acc, sem0, sem1) = refs
    c = lax.axis_index("c")
    s = lax.axis_index("s")
    row0 = s * RPT

    def per_tile(fn):
      @pl.when(s < NS - 1)
      def _():
        fn(row0, RPT)

      @pl.when(s == NS - 1)
      def _():
        fn((NS - 1) * RPT, RPT_LAST)

    per_tile(lambda o, n: pltpu.sync_copy(zrow.at[pl.ds(0, n)],
                                          acc.at[pl.ds(o, n)]))
    if with_counts:
      per_tile(lambda o, n: pltpu.sync_copy(zcnt.at[pl.ds(0, n)],
                                            cacc.at[pl.ds(o, n)]))
      pltpu.sync_copy(ones_h, ones_v)
    plsc.subcore_barrier()

    def accum(dst_v, rows, g):
      pltpu.sync_copy(rows, acc.at[dst_v.at[g]], add=True)
      if with_counts:
        pltpu.sync_copy(ones_v, cacc.at[dst_v.at[g]], add=True)

    def body(j, carry):
      g0 = 2 * j
      h1 = pltpu.async_copy(t_cat.at[src_v.at[g0 + 1]], rows1, sem1)
      pltpu.make_async_copy(t_cat.at[src_v.at[g0]], rows0, sem0).wait()
      accum(dst_v, rows0, g0)

      @pl.when(j < CHG // 2 - 1)
      def _():
        pltpu.async_copy(t_cat.at[src_v.at[g0 + 2]], rows0, sem0)
      h1.wait()
      accum(dst_v, rows1, g0 + 1)
      return carry

    for ci in range(NCH):
      pltpu.sync_copy(e_src.at[c, s, pl.ds(ci * CHG, CHG)], src_v)
      pltpu.sync_copy(e_dst.at[c, s, pl.ds(ci * CHG, CHG)], dst_v)
      pltpu.async_copy(t_cat.at[src_v.at[0]], rows0, sem0)
      lax.fori_loop(0, CHG // 2, body, 0)

    plsc.subcore_barrier()
    per_tile(lambda o, n: pltpu.sync_copy(acc.at[pl.ds(o, n)],
                                          agg_out.at[c, pl.ds(o, n)]))
    if with_counts:
      per_tile(lambda o, n: pltpu.sync_copy(cacc.at[pl.ds(o, n)],
                                            cnt_out.at[c, pl.ds(o, n)]))

  return spmm


_spmm_counts = _make_spmm(True)
_spmm_plain = _make_spmm(False)


@functools.partial(
    pl.kernel,
    out_type=jax.ShapeDtypeStruct((2, BPAD, D), jnp.float32),
    mesh=_mesh,
    scratch_types=[
        pltpu.VMEM((LNG, LG), jnp.int32),
        pltpu.VMEM((LG, D), jnp.float32),
        pltpu.VMEM((LG, D), jnp.float32),
        pltpu.SemaphoreType.DMA,
        pltpu.SemaphoreType.DMA,
    ])
def _link_gather(u_cat, idx, z_out, idx_v, rows0, rows1, sem0, sem1):
  c = lax.axis_index("c")
  s = lax.axis_index("s")
  w = s * NC + c
  for t in range(2):
    pltpu.sync_copy(idx.at[t, w], idx_v)
    pltpu.async_copy(u_cat.at[idx_v.at[0]], rows0, sem0)

    def body(j, carry):
      g0 = 2 * j
      h1 = pltpu.async_copy(u_cat.at[idx_v.at[g0 + 1]], rows1, sem1)
      pltpu.make_async_copy(u_cat.at[idx_v.at[g0]], rows0, sem0).wait()
      pltpu.sync_copy(rows0, z_out.at[t, pl.ds(w * LPW + g0 * LG, LG)])

      @pl.when(j < LNG // 2 - 1)
      def _():
        pltpu.async_copy(u_cat.at[idx_v.at[g0 + 2]], rows0, sem0)
      h1.wait()
      pltpu.sync_copy(rows1, z_out.at[t, pl.ds(w * LPW + (g0 + 1) * LG, LG)])
      return carry
    lax.fori_loop(0, LNG // 2, body, 0)


def _pre0_body(xd, xs, w0, w1, t_ref):
  t_ref[0] = jnp.dot(xd[...], w0[...], preferred_element_type=jnp.float32)
  t_ref[1] = jnp.dot(xs[...], w1[...], preferred_element_type=jnp.float32)


def _finish(agg, cnt, rt):
  mean = agg / jnp.maximum(cnt[:, 0:1], 1.0)
  o = mean + rt
  nrm = jnp.sqrt(jnp.sum(o * o, axis=-1, keepdims=True))
  return o / jnp.maximum(nrm, 1e-12)


def _post0_body(agg, cnt, xd, xs, wr0a, wr0b, b0a, b0b,
                wn1a, wn1b, wr1a, wr1b, b1a, b1b, t1_ref, r1_ref):
  f32 = jnp.float32
  di1 = _finish(agg[0], cnt[0],
                jnp.dot(xs[...], wr0a[...], preferred_element_type=f32)
                + b0a[...])
  dr1 = _finish(agg[1], cnt[1],
                jnp.dot(xd[...], wr0b[...], preferred_element_type=f32)
                + b0b[...])
  di1 = jnp.maximum(di1, 0.0)
  dr1 = jnp.maximum(dr1, 0.0)
  t1_ref[0] = jnp.dot(dr1, wn1a[...], preferred_element_type=f32)
  t1_ref[1] = jnp.dot(di1, wn1b[...], preferred_element_type=f32)
  r1_ref[0] = jnp.dot(di1, wr1a[...], preferred_element_type=f32) + b1a[...]
  r1_ref[1] = jnp.dot(dr1, wr1b[...], preferred_element_type=f32) + b1b[...]


def _post1_body(agg, cnt, r1, wt, wb, bl, u_ref):
  f32 = jnp.float32
  di2 = _finish(agg[0], cnt[0], r1[0])
  dr2 = _finish(agg[1], cnt[1], r1[1])
  u_ref[0] = jnp.dot(dr2, wt[...], preferred_element_type=f32) + bl[...]
  u_ref[1] = jnp.dot(di2, wb[...], preferred_element_type=f32)


def _score_body(z, w2, b2, o_ref):
  sc = jnp.sum(jnp.maximum(z[0] + z[1], 0.0) * w2[...], axis=-1)
  o_ref[...] = sc.reshape(SB // D, D) + b2[...]


def _node_spec(stacked):
  if stacked:
    return pl.BlockSpec((2, RB, D), lambda i: (0, i, 0))
  return pl.BlockSpec((RB, D), lambda i: (i, 0))


def _full_spec(shape):
  return pl.BlockSpec(shape, lambda i: tuple(0 for _ in shape))


def kernel(x_drug, x_disease, edge_d2s, edge_s2d, edge_label_index,
           W_nb_0_d2s, b_nb_0_d2s, W_rt_0_d2s,
           W_nb_0_s2d, b_nb_0_s2d, W_rt_0_s2d,
           W_nb_1_d2s, b_nb_1_d2s, W_rt_1_d2s,
           W_nb_1_s2d, b_nb_1_s2d, W_rt_1_s2d,
           W_lin1, b_lin1, W_lin2, b_lin2):
  f32 = jnp.float32
  grid_n = N // RB

  t0 = pl.pallas_call(
      _pre0_body,
      grid=(grid_n,),
      in_specs=[_node_spec(False), _node_spec(False),
                _full_spec((D, D)), _full_spec((D, D))],
      out_specs=_node_spec(True),
      out_shape=jax.ShapeDtypeStruct((2, N, D), f32),
  )(x_drug, x_disease, W_nb_0_d2s, W_nb_0_s2d)

  src_cat = jnp.stack([edge_d2s[0], edge_s2d[0] + N]).reshape(2, NS, NG, G)
  dst_all = jnp.stack([edge_d2s[1], edge_s2d[1]]).reshape(2, NS, NG, G)
  zrow = jnp.zeros((RPT_LAST, D), f32)
  zcnt = jnp.zeros((RPT_LAST, CW), f32)
  ones_g = jnp.ones((G, CW), f32)

  agg0, cnt = _spmm_counts(t0.reshape(2 * N, D), src_cat, dst_all,
                           zrow, zcnt, ones_g)

  b_shapes = lambda b: b.reshape(1, D)
  t1, r1 = pl.pallas_call(
      _post0_body,
      grid=(grid_n,),
      in_specs=[_node_spec(True),
                pl.BlockSpec((2, RB, CW), lambda i: (0, i, 0)),
                _node_spec(False), _node_spec(False),
                _full_spec((D, D)), _full_spec((D, D)),
                _full_spec((1, D)), _full_spec((1, D)),
                _full_spec((D, D)), _full_spec((D, D)),
                _full_spec((D, D)), _full_spec((D, D)),
                _full_spec((1, D)), _full_spec((1, D))],
      out_specs=[_node_spec(True), _node_spec(True)],
      out_shape=[jax.ShapeDtypeStruct((2, N, D), f32),
                 jax.ShapeDtypeStruct((2, N, D), f32)],
  )(agg0, cnt, x_drug, x_disease,
    W_rt_0_d2s, W_rt_0_s2d, b_shapes(b_nb_0_d2s), b_shapes(b_nb_0_s2d),
    W_nb_1_d2s, W_nb_1_s2d, W_rt_1_d2s, W_rt_1_s2d,
    b_shapes(b_nb_1_d2s), b_shapes(b_nb_1_s2d))

  agg1, = _spmm_plain(t1.reshape(2 * N, D), src_cat, dst_all, zrow)

  u = pl.pallas_call(
      _post1_body,
      grid=(grid_n,),
      in_specs=[_node_spec(True),
                pl.BlockSpec((2, RB, CW), lambda i: (0, i, 0)),
                _node_spec(True),
                _full_spec((D, D)), _full_spec((D, D)), _full_spec((1, D))],
      out_specs=_node_spec(True),
      out_shape=jax.ShapeDtypeStruct((2, N, D), f32),
  )(agg1, cnt, r1, W_lin1[:D], W_lin1[D:], b_lin1.reshape(1, D))

  npad = BPAD - B
  row_pad = jnp.concatenate([edge_label_index[0],
                             jnp.arange(npad, dtype=jnp.int32)])
  col_pad = jnp.concatenate([edge_label_index[1],
                             jnp.arange(npad, dtype=jnp.int32)])
  idx_link = jnp.stack([row_pad, col_pad + N]).reshape(2, NC * NS, LNG, LG)
  z = _link_gather(u.reshape(2 * N, D), idx_link)

  scores = pl.pallas_call(
      _score_body,
      grid=(BPAD // SB,),
      in_specs=[pl.BlockSpec((2, SB, D), lambda i: (0, i, 0)),
                _full_spec((1, D)), _full_spec((1, D))],
      out_specs=pl.BlockSpec((SB // D, D), lambda i: (i, 0)),
      out_shape=jax.ShapeDtypeStruct((BPAD // D, D), f32),
  )(z, W_lin2.reshape(1, D), jnp.broadcast_to(b_lin2.reshape(1, 1), (1, D)))

  return scores.reshape(-1)[:B]


# speedup vs baseline: 1.0436x; 1.0362x over previous
import functools

import jax
import jax.numpy as jnp
from jax import lax
from jax.experimental import pallas as pl
from jax.experimental.pallas import tpu as pltpu
from jax.experimental.pallas import tpu_sc as plsc

N = 10000
E = 320000
D = 128
B = 100000
BPAD = 102400
NC, NS = 2, 16
EPT = E // NS
G = 100
NG = EPT // G
RPT = 624
RPT_LAST = N - 15 * RPT
CW = 16
LG = 80
LPW = BPAD // (NC * NS)
LNG = LPW // LG
RB = 1000
SB = 2048

_mesh = plsc.VectorSubcoreMesh(core_axis_name="c", subcore_axis_name="s")


NCH = 5
CHG = NG // NCH

_sc_params = pltpu.CompilerParams(use_tc_tiling_on_sc=False)


def _make_spmm(with_counts):
  out_type = [jax.ShapeDtypeStruct((2, N, D), jnp.float32)]
  scratch = [
      pltpu.VMEM((CHG, G), jnp.int32),
      pltpu.VMEM((CHG, G), jnp.int32),
      pltpu.VMEM((G, D), jnp.float32),
      pltpu.VMEM((G, D), jnp.float32),
      pltpu.VMEM_SHARED((N, D), jnp.float32),
      pltpu.SemaphoreType.DMA,
      pltpu.SemaphoreType.DMA,
  ]
  if with_counts:
    out_type.append(jax.ShapeDtypeStruct((2, N, CW), jnp.float32))
    scratch += [
        pltpu.VMEM((G, CW), jnp.float32),
        pltpu.VMEM_SHARED((N, CW), jnp.float32),
    ]

  @functools.partial(pl.kernel, out_type=out_type, mesh=_mesh,
                     scratch_types=scratch, compiler_params=_sc_params)
  def spmm(*refs):
    if with_counts:
      (t_cat, e_src, e_dst, zrow, zcnt, ones_h,
       agg_out, cnt_out, src_v, dst_v, rows0, rows1, acc,
       sem0, sem1, ones_v, cacc) = refs
    else:
      (t_cat, e_src, e_dst, zrow,
       agg_out, src_v, dst_v, rows0, rows1, acc, sem0, sem1) = refs
    c = lax.axis_index("c")
    s = lax.axis_index("s")
    row0 = s * RPT

    def per_tile(fn):
      @pl.when(s < NS - 1)
      def _():
        fn(row0, RPT)

      @pl.when(s == NS - 1)
      def _():
        fn((NS - 1) * RPT, RPT_LAST)

    per_tile(lambda o, n: pltpu.sync_copy(zrow.at[pl.ds(0, n)],
                                          acc.at[pl.ds(o, n)]))
    if with_counts:
      per_tile(lambda o, n: pltpu.sync_copy(zcnt.at[pl.ds(0, n)],
                                            cacc.at[pl.ds(o, n)]))
      pltpu.sync_copy(ones_h, ones_v)
    plsc.subcore_barrier()

    def accum(dst_v, rows, g):
      pltpu.sync_copy(rows, acc.at[dst_v.at[g]], add=True)
      if with_counts:
        pltpu.sync_copy(ones_v, cacc.at[dst_v.at[g]], add=True)

    def body(j, carry):
      g0 = 2 * j
      h1 = pltpu.async_copy(t_cat.at[src_v.at[g0 + 1]], rows1, sem1)
      pltpu.make_async_copy(t_cat.at[src_v.at[g0]], rows0, sem0).wait()
      accum(dst_v, rows0, g0)

      @pl.when(j < CHG // 2 - 1)
      def _():
        pltpu.async_copy(t_cat.at[src_v.at[g0 + 2]], rows0, sem0)
      h1.wait()
      accum(dst_v, rows1, g0 + 1)
      return carry

    for ci in range(NCH):
      pltpu.sync_copy(e_src.at[c, s, pl.ds(ci * CHG, CHG)], src_v)
      pltpu.sync_copy(e_dst.at[c, s, pl.ds(ci * CHG, CHG)], dst_v)
      pltpu.async_copy(t_cat.at[src_v.at[0]], rows0, sem0)
      lax.fori_loop(0, CHG // 2, body, 0)

    plsc.subcore_barrier()
    per_tile(lambda o, n: pltpu.sync_copy(acc.at[pl.ds(o, n)],
                                          agg_out.at[c, pl.ds(o, n)]))
    if with_counts:
      per_tile(lambda o, n: pltpu.sync_copy(cacc.at[pl.ds(o, n)],
                                            cnt_out.at[c, pl.ds(o, n)]))

  return spmm


_spmm_counts = _make_spmm(True)
_spmm_plain = _make_spmm(False)


@functools.partial(
    pl.kernel,
    out_type=jax.ShapeDtypeStruct((2, BPAD, D), jnp.float32),
    mesh=_mesh,
    scratch_types=[
        pltpu.VMEM((LNG, LG), jnp.int32),
        pltpu.VMEM((LG, D), jnp.float32),
        pltpu.VMEM((LG, D), jnp.float32),
        pltpu.SemaphoreType.DMA,
        pltpu.SemaphoreType.DMA,
    ])
def _link_gather(u_cat, idx, z_out, idx_v, rows0, rows1, sem0, sem1):
  c = lax.axis_index("c")
  s = lax.axis_index("s")
  w = s * NC + c
  for t in range(2):
    pltpu.sync_copy(idx.at[t, w], idx_v)
    pltpu.async_copy(u_cat.at[idx_v.at[0]], rows0, sem0)

    def body(j, carry):
      g0 = 2 * j
      h1 = pltpu.async_copy(u_cat.at[idx_v.at[g0 + 1]], rows1, sem1)
      pltpu.make_async_copy(u_cat.at[idx_v.at[g0]], rows0, sem0).wait()
      pltpu.sync_copy(rows0, z_out.at[t, pl.ds(w * LPW + g0 * LG, LG)])

      @pl.when(j < LNG // 2 - 1)
      def _():
        pltpu.async_copy(u_cat.at[idx_v.at[g0 + 2]], rows0, sem0)
      h1.wait()
      pltpu.sync_copy(rows1, z_out.at[t, pl.ds(w * LPW + (g0 + 1) * LG, LG)])
      return carry
    lax.fori_loop(0, LNG // 2, body, 0)


def _pre0_body(xd, xs, w0, w1, t_ref):
  t_ref[0] = jnp.dot(xd[...], w0[...], preferred_element_type=jnp.float32)
  t_ref[1] = jnp.dot(xs[...], w1[...], preferred_element_type=jnp.float32)


def _finish(agg, cnt, rt):
  mean = agg / jnp.maximum(cnt[:, 0:1], 1.0)
  o = mean + rt
  nrm = jnp.sqrt(jnp.sum(o * o, axis=-1, keepdims=True))
  return o / jnp.maximum(nrm, 1e-12)


def _post0_body(agg, cnt, xd, xs, wr0a, wr0b, b0a, b0b,
                wn1a, wn1b, wr1a, wr1b, b1a, b1b, t1_ref, r1_ref):
  f32 = jnp.float32
  di1 = _finish(agg[0], cnt[0],
                jnp.dot(xs[...], wr0a[...], preferred_element_type=f32)
                + b0a[...])
  dr1 = _finish(agg[1], cnt[1],
                jnp.dot(xd[...], wr0b[...], preferred_element_type=f32)
                + b0b[...])
  di1 = jnp.maximum(di1, 0.0)
  dr1 = jnp.maximum(dr1, 0.0)
  t1_ref[0] = jnp.dot(dr1, wn1a[...], preferred_element_type=f32)
  t1_ref[1] = jnp.dot(di1, wn1b[...], preferred_element_type=f32)
  r1_ref[0] = jnp.dot(di1, wr1a[...], preferred_element_type=f32) + b1a[...]
  r1_ref[1] = jnp.dot(dr1, wr1b[...], preferred_element_type=f32) + b1b[...]


def _post1_body(agg, cnt, r1, wt, wb, bl, u_ref):
  f32 = jnp.float32
  di2 = _finish(agg[0], cnt[0], r1[0])
  dr2 = _finish(agg[1], cnt[1], r1[1])
  u_ref[0] = jnp.dot(dr2, wt[...], preferred_element_type=f32) + bl[...]
  u_ref[1] = jnp.dot(di2, wb[...], preferred_element_type=f32)


def _score_body(z, w2, b2, o_ref):
  sc = jnp.sum(jnp.maximum(z[0] + z[1], 0.0) * w2[...], axis=-1)
  o_ref[...] = sc.reshape(SB // D, D) + b2[...]


def _node_spec(stacked):
  if stacked:
    return pl.BlockSpec((2, RB, D), lambda i: (0, i, 0))
  return pl.BlockSpec((RB, D), lambda i: (i, 0))


def _full_spec(shape):
  return pl.BlockSpec(shape, lambda i: tuple(0 for _ in shape))


def kernel(x_drug, x_disease, edge_d2s, edge_s2d, edge_label_index,
           W_nb_0_d2s, b_nb_0_d2s, W_rt_0_d2s,
           W_nb_0_s2d, b_nb_0_s2d, W_rt_0_s2d,
           W_nb_1_d2s, b_nb_1_d2s, W_rt_1_d2s,
           W_nb_1_s2d, b_nb_1_s2d, W_rt_1_s2d,
           W_lin1, b_lin1, W_lin2, b_lin2):
  f32 = jnp.float32
  grid_n = N // RB

  t0 = pl.pallas_call(
      _pre0_body,
      grid=(grid_n,),
      in_specs=[_node_spec(False), _node_spec(False),
                _full_spec((D, D)), _full_spec((D, D))],
      out_specs=_node_spec(True),
      out_shape=jax.ShapeDtypeStruct((2, N, D), f32),
  )(x_drug, x_disease, W_nb_0_d2s, W_nb_0_s2d)

  src_cat = jnp.stack([edge_d2s[0], edge_s2d[0] + N]).reshape(2, NS, NG, G)
  dst_all = jnp.stack([edge_d2s[1], edge_s2d[1]]).reshape(2, NS, NG, G)
  zrow = jnp.zeros((RPT_LAST, D), f32)
  zcnt = jnp.zeros((RPT_LAST, CW), f32)
  ones_g = jnp.ones((G, CW), f32)

  agg0, cnt = _spmm_counts(t0.reshape(2 * N, D), src_cat, dst_all,
                           zrow, zcnt, ones_g)

  b_shapes = lambda b: b.reshape(1, D)
  t1, r1 = pl.pallas_call(
      _post0_body,
      grid=(grid_n,),
      in_specs=[_node_spec(True),
                pl.BlockSpec((2, RB, CW), lambda i: (0, i, 0)),
                _node_spec(False), _node_spec(False),
                _full_spec((D, D)), _full_spec((D, D)),
                _full_spec((1, D)), _full_spec((1, D)),
                _full_spec((D, D)), _full_spec((D, D)),
                _full_spec((D, D)), _full_spec((D, D)),
                _full_spec((1, D)), _full_spec((1, D))],
      out_specs=[_node_spec(True), _node_spec(True)],
      out_shape=[jax.ShapeDtypeStruct((2, N, D), f32),
                 jax.ShapeDtypeStruct((2, N, D), f32)],
  )(agg0, cnt, x_drug, x_disease,
    W_rt_0_d2s, W_rt_0_s2d, b_shapes(b_nb_0_d2s), b_shapes(b_nb_0_s2d),
    W_nb_1_d2s, W_nb_1_s2d, W_rt_1_d2s, W_rt_1_s2d,
    b_shapes(b_nb_1_d2s), b_shapes(b_nb_1_s2d))

  agg1, = _spmm_plain(t1.reshape(2 * N, D), src_cat, dst_all, zrow)

  u = pl.pallas_call(
      _post1_body,
      grid=(grid_n,),
      in_specs=[_node_spec(True),
                pl.BlockSpec((2, RB, CW), lambda i: (0, i, 0)),
                _node_spec(True),
                _full_spec((D, D)), _full_spec((D, D)), _full_spec((1, D))],
      out_specs=_node_spec(True),
      out_shape=jax.ShapeDtypeStruct((2, N, D), f32),
  )(agg1, cnt, r1, W_lin1[:D], W_lin1[D:], b_lin1.reshape(1, D))

  npad = BPAD - B
  row_pad = jnp.concatenate([edge_label_index[0],
                             jnp.arange(npad, dtype=jnp.int32)])
  col_pad = jnp.concatenate([edge_label_index[1],
                             jnp.arange(npad, dtype=jnp.int32)])
  idx_link = jnp.stack([row_pad, col_pad + N]).reshape(2, NC * NS, LNG, LG)
  z = _link_gather(u.reshape(2 * N, D), idx_link)

  scores = pl.pallas_call(
      _score_body,
      grid=(BPAD // SB,),
      in_specs=[pl.BlockSpec((2, SB, D), lambda i: (0, i, 0)),
                _full_spec((1, D)), _full_spec((1, D))],
      out_specs=pl.BlockSpec((SB // D, D), lambda i: (i, 0)),
      out_shape=jax.ShapeDtypeStruct((BPAD // D, D), f32),
  )(z, W_lin2.reshape(1, D), jnp.broadcast_to(b_lin2.reshape(1, 1), (1, D)))

  return scores.reshape(-1)[:B]


# speedup vs baseline: 1.0775x; 1.0325x over previous
import functools

import jax
import jax.numpy as jnp
from jax import lax
from jax.experimental import pallas as pl
from jax.experimental.pallas import tpu as pltpu
from jax.experimental.pallas import tpu_sc as plsc

N = 10000
E = 320000
D = 128
B = 100000
BPAD = 102400
NC, NS = 2, 16
EPT = E // NS
G = 100
NG = EPT // G
RPT = 624
RPT_LAST = N - 15 * RPT
CW = 16
LG = 80
LPW = BPAD // (NC * NS)
LNG = LPW // LG
RB = 2000
SB = 4096

_mesh = plsc.VectorSubcoreMesh(core_axis_name="c", subcore_axis_name="s")


NCH = 5
CHG = NG // NCH

_sc_params = pltpu.CompilerParams(use_tc_tiling_on_sc=False)


def _make_spmm(with_counts):
  out_type = [jax.ShapeDtypeStruct((2, N, D), jnp.float32)]
  scratch = [
      pltpu.VMEM((CHG, G), jnp.int32),
      pltpu.VMEM((CHG, G), jnp.int32),
      pltpu.VMEM((G, D), jnp.float32),
      pltpu.VMEM((G, D), jnp.float32),
      pltpu.VMEM_SHARED((N, D), jnp.float32),
      pltpu.SemaphoreType.DMA,
      pltpu.SemaphoreType.DMA,
  ]
  if with_counts:
    out_type.append(jax.ShapeDtypeStruct((2, N, CW), jnp.float32))
    scratch += [
        pltpu.VMEM((G, CW), jnp.float32),
        pltpu.VMEM_SHARED((N, CW), jnp.float32),
    ]

  @functools.partial(pl.kernel, out_type=out_type, mesh=_mesh,
                     scratch_types=scratch, compiler_params=_sc_params)
  def spmm(*refs):
    if with_counts:
      (t_cat, e_src, e_dst, zrow, zcnt, ones_h,
       agg_out, cnt_out, src_v, dst_v, rows0, rows1, acc,
       sem0, sem1, ones_v, cacc) = refs
    else:
      (t_cat, e_src, e_dst, zrow,
       agg_out, src_v, dst_v, rows0, rows1, acc, sem0, sem1) = refs
    c = lax.axis_index("c")
    s = lax.axis_index("s")
    row0 = s * RPT

    def per_tile(fn):
      @pl.when(s < NS - 1)
      def _():
        fn(row0, RPT)

      @pl.when(s == NS - 1)
      def _():
        fn((NS - 1) * RPT, RPT_LAST)

    per_tile(lambda o, n: pltpu.sync_copy(zrow.at[pl.ds(0, n)],
                                          acc.at[pl.ds(o, n)]))
    if with_counts:
      per_tile(lambda o, n: pltpu.sync_copy(zcnt.at[pl.ds(0, n)],
                                            cacc.at[pl.ds(o, n)]))
      pltpu.sync_copy(ones_h, ones_v)
    plsc.subcore_barrier()

    def accum(dst_v, rows, g):
      pltpu.sync_copy(rows, acc.at[dst_v.at[g]], add=True)
      if with_counts:
        pltpu.sync_copy(ones_v, cacc.at[dst_v.at[g]], add=True)

    def body(j, carry):
      g0 = 2 * j
      h1 = pltpu.async_copy(t_cat.at[src_v.at[g0 + 1]], rows1, sem1)
      pltpu.make_async_copy(t_cat.at[src_v.at[g0]], rows0, sem0).wait()
      accum(dst_v, rows0, g0)

      @pl.when(j < CHG // 2 - 1)
      def _():
        pltpu.async_copy(t_cat.at[src_v.at[g0 + 2]], rows0, sem0)
      h1.wait()
      accum(dst_v, rows1, g0 + 1)
      return carry

    for ci in range(NCH):
      pltpu.sync_copy(e_src.at[c, s, pl.ds(ci * CHG, CHG)], src_v)
      pltpu.sync_copy(e_dst.at[c, s, pl.ds(ci * CHG, CHG)], dst_v)
      pltpu.async_copy(t_cat.at[src_v.at[0]], rows0, sem0)
      lax.fori_loop(0, CHG // 2, body, 0)

    plsc.subcore_barrier()
    per_tile(lambda o, n: pltpu.sync_copy(acc.at[pl.ds(o, n)],
                                          agg_out.at[c, pl.ds(o, n)]))
    if with_counts:
      per_tile(lambda o, n: pltpu.sync_copy(cacc.at[pl.ds(o, n)],
                                            cnt_out.at[c, pl.ds(o, n)]))

  return spmm


_spmm_counts = _make_spmm(True)
_spmm_plain = _make_spmm(False)


@functools.partial(
    pl.kernel,
    out_type=jax.ShapeDtypeStruct((2, BPAD, D), jnp.float32),
    mesh=_mesh,
    scratch_types=[
        pltpu.VMEM((LNG, LG), jnp.int32),
        pltpu.VMEM((LG, D), jnp.float32),
        pltpu.VMEM((LG, D), jnp.float32),
        pltpu.SemaphoreType.DMA,
        pltpu.SemaphoreType.DMA,
    ])
def _link_gather(u_cat, idx, z_out, idx_v, rows0, rows1, sem0, sem1):
  c = lax.axis_index("c")
  s = lax.axis_index("s")
  w = s * NC + c
  for t in range(2):
    pltpu.sync_copy(idx.at[t, w], idx_v)
    pltpu.async_copy(u_cat.at[idx_v.at[0]], rows0, sem0)

    def body(j, carry):
      g0 = 2 * j
      h1 = pltpu.async_copy(u_cat.at[idx_v.at[g0 + 1]], rows1, sem1)
      pltpu.make_async_copy(u_cat.at[idx_v.at[g0]], rows0, sem0).wait()
      pltpu.sync_copy(rows0, z_out.at[t, pl.ds(w * LPW + g0 * LG, LG)])

      @pl.when(j < LNG // 2 - 1)
      def _():
        pltpu.async_copy(u_cat.at[idx_v.at[g0 + 2]], rows0, sem0)
      h1.wait()
      pltpu.sync_copy(rows1, z_out.at[t, pl.ds(w * LPW + (g0 + 1) * LG, LG)])
      return carry
    lax.fori_loop(0, LNG // 2, body, 0)


def _pre0_body(xd, xs, w0, w1, t_ref):
  t_ref[0] = jnp.dot(xd[...], w0[...], preferred_element_type=jnp.float32)
  t_ref[1] = jnp.dot(xs[...], w1[...], preferred_element_type=jnp.float32)


def _finish(agg, cnt, rt):
  mean = agg / jnp.maximum(cnt[:, 0:1], 1.0)
  o = mean + rt
  nrm = jnp.sqrt(jnp.sum(o * o, axis=-1, keepdims=True))
  return o / jnp.maximum(nrm, 1e-12)


def _post0_body(agg, cnt, xd, xs, wr0a, wr0b, b0a, b0b,
                wn1a, wn1b, wr1a, wr1b, b1a, b1b, t1_ref, r1_ref):
  f32 = jnp.float32
  di1 = _finish(agg[0], cnt[0],
                jnp.dot(xs[...], wr0a[...], preferred_element_type=f32)
                + b0a[...])
  dr1 = _finish(agg[1], cnt[1],
                jnp.dot(xd[...], wr0b[...], preferred_element_type=f32)
                + b0b[...])
  di1 = jnp.maximum(di1, 0.0)
  dr1 = jnp.maximum(dr1, 0.0)
  t1_ref[0] = jnp.dot(dr1, wn1a[...], preferred_element_type=f32)
  t1_ref[1] = jnp.dot(di1, wn1b[...], preferred_element_type=f32)
  r1_ref[0] = jnp.dot(di1, wr1a[...], preferred_element_type=f32) + b1a[...]
  r1_ref[1] = jnp.dot(dr1, wr1b[...], preferred_element_type=f32) + b1b[...]


def _post1_body(agg, cnt, r1, wt, wb, bl, u_ref):
  f32 = jnp.float32
  di2 = _finish(agg[0], cnt[0], r1[0])
  dr2 = _finish(agg[1], cnt[1], r1[1])
  u_ref[0] = jnp.dot(dr2, wt[...], preferred_element_type=f32) + bl[...]
  u_ref[1] = jnp.dot(di2, wb[...], preferred_element_type=f32)


def _score_body(z, w2, b2, o_ref):
  sc = jnp.sum(jnp.maximum(z[0] + z[1], 0.0) * w2[...], axis=-1)
  o_ref[...] = sc.reshape(SB // D, D) + b2[...]


def _node_spec(stacked):
  if stacked:
    return pl.BlockSpec((2, RB, D), lambda i: (0, i, 0))
  return pl.BlockSpec((RB, D), lambda i: (i, 0))


def _full_spec(shape):
  return pl.BlockSpec(shape, lambda i: tuple(0 for _ in shape))


def kernel(x_drug, x_disease, edge_d2s, edge_s2d, edge_label_index,
           W_nb_0_d2s, b_nb_0_d2s, W_rt_0_d2s,
           W_nb_0_s2d, b_nb_0_s2d, W_rt_0_s2d,
           W_nb_1_d2s, b_nb_1_d2s, W_rt_1_d2s,
           W_nb_1_s2d, b_nb_1_s2d, W_rt_1_s2d,
           W_lin1, b_lin1, W_lin2, b_lin2):
  f32 = jnp.float32
  grid_n = N // RB

  t0 = pl.pallas_call(
      _pre0_body,
      grid=(grid_n,),
      in_specs=[_node_spec(False), _node_spec(False),
                _full_spec((D, D)), _full_spec((D, D))],
      out_specs=_node_spec(True),
      out_shape=jax.ShapeDtypeStruct((2, N, D), f32),
  )(x_drug, x_disease, W_nb_0_d2s, W_nb_0_s2d)

  src_cat = jnp.stack([edge_d2s[0], edge_s2d[0] + N]).reshape(2, NS, NG, G)
  dst_all = jnp.stack([edge_d2s[1], edge_s2d[1]]).reshape(2, NS, NG, G)
  zrow = jnp.zeros((RPT_LAST, D), f32)
  zcnt = jnp.zeros((RPT_LAST, CW), f32)
  ones_g = jnp.ones((G, CW), f32)

  agg0, cnt = _spmm_counts(t0.reshape(2 * N, D), src_cat, dst_all,
                           zrow, zcnt, ones_g)

  b_shapes = lambda b: b.reshape(1, D)
  t1, r1 = pl.pallas_call(
      _post0_body,
      grid=(grid_n,),
      in_specs=[_node_spec(True),
                pl.BlockSpec((2, RB, CW), lambda i: (0, i, 0)),
                _node_spec(False), _node_spec(False),
                _full_spec((D, D)), _full_spec((D, D)),
                _full_spec((1, D)), _full_spec((1, D)),
                _full_spec((D, D)), _full_spec((D, D)),
                _full_spec((D, D)), _full_spec((D, D)),
                _full_spec((1, D)), _full_spec((1, D))],
      out_specs=[_node_spec(True), _node_spec(True)],
      out_shape=[jax.ShapeDtypeStruct((2, N, D), f32),
                 jax.ShapeDtypeStruct((2, N, D), f32)],
  )(agg0, cnt, x_drug, x_disease,
    W_rt_0_d2s, W_rt_0_s2d, b_shapes(b_nb_0_d2s), b_shapes(b_nb_0_s2d),
    W_nb_1_d2s, W_nb_1_s2d, W_rt_1_d2s, W_rt_1_s2d,
    b_shapes(b_nb_1_d2s), b_shapes(b_nb_1_s2d))

  agg1, = _spmm_plain(t1.reshape(2 * N, D), src_cat, dst_all, zrow)

  u = pl.pallas_call(
      _post1_body,
      grid=(grid_n,),
      in_specs=[_node_spec(True),
                pl.BlockSpec((2, RB, CW), lambda i: (0, i, 0)),
                _node_spec(True),
                _full_spec((D, D)), _full_spec((D, D)), _full_spec((1, D))],
      out_specs=_node_spec(True),
      out_shape=jax.ShapeDtypeStruct((2, N, D), f32),
  )(agg1, cnt, r1, W_lin1[:D], W_lin1[D:], b_lin1.reshape(1, D))

  npad = BPAD - B
  row_pad = jnp.concatenate([edge_label_index[0],
                             jnp.arange(npad, dtype=jnp.int32)])
  col_pad = jnp.concatenate([edge_label_index[1],
                             jnp.arange(npad, dtype=jnp.int32)])
  idx_link = jnp.stack([row_pad, col_pad + N]).reshape(2, NC * NS, LNG, LG)
  z = _link_gather(u.reshape(2 * N, D), idx_link)

  scores = pl.pallas_call(
      _score_body,
      grid=(BPAD // SB,),
      in_specs=[pl.BlockSpec((2, SB, D), lambda i: (0, i, 0)),
                _full_spec((1, D)), _full_spec((1, D))],
      out_specs=pl.BlockSpec((SB // D, D), lambda i: (i, 0)),
      out_shape=jax.ShapeDtypeStruct((BPAD // D, D), f32),
  )(z, W_lin2.reshape(1, D), jnp.broadcast_to(b_lin2.reshape(1, 1), (1, D)))

  return scores.reshape(-1)[:B]
